# initial kernel scaffold (unmeasured)
import jax
import jax.numpy as jnp
from jax import lax
from jax.experimental import pallas as pl
from jax.experimental.pallas import tpu as pltpu

N_DEV = 16


def kernel(x, w_mat):
    m_per, k = x.shape
    _, n_per = w_mat.shape
    m_glob = N_DEV * m_per

    def body(x_ref, w_ref, out_ref, comm_ref, send_sems, recv_sems, credit_sem):
        my = lax.axis_index("i")
        left = (my - 1) % N_DEV
        right = (my + 1) % N_DEV

        comm_ref[0] = x_ref[...]
        out_ref[pl.ds(my * m_per, m_per), :] = jnp.dot(
            x_ref[...], w_ref[...], preferred_element_type=jnp.float32
        )

        for h in range(N_DEV - 1):
            ss = h % 2
            rs = (h + 1) % 2
            if h == 2:
                pl.semaphore_wait(credit_sem, 2)
            elif h > 2:
                pl.semaphore_wait(credit_sem, 1)

            rdma = pltpu.make_async_remote_copy(
                src_ref=comm_ref.at[ss],
                dst_ref=comm_ref.at[rs],
                send_sem=send_sems.at[ss],
                recv_sem=recv_sems.at[rs],
                device_id=(right,),
                device_id_type=pl.DeviceIdType.MESH,
            )
            rdma.start()
            rdma.wait()

            origin = (my - h - 1) % N_DEV
            out_ref[pl.ds(origin * m_per, m_per), :] = jnp.dot(
                comm_ref[rs], w_ref[...], preferred_element_type=jnp.float32
            )
            if h < N_DEV - 2:
                pl.semaphore_signal(
                    credit_sem,
                    inc=1,
                    device_id=(left,),
                    device_id_type=pl.DeviceIdType.MESH,
                )

    return pl.pallas_call(
        body,
        out_shape=jax.ShapeDtypeStruct((m_glob, n_per), jnp.float32),
        in_specs=[
            pl.BlockSpec(memory_space=pltpu.VMEM),
            pl.BlockSpec(memory_space=pltpu.VMEM),
        ],
        out_specs=pl.BlockSpec(memory_space=pltpu.VMEM),
        scratch_shapes=[
            pltpu.VMEM((2, m_per, k), jnp.float32),
            pltpu.SemaphoreType.DMA((2,)),
            pltpu.SemaphoreType.DMA((2,)),
            pltpu.SemaphoreType.REGULAR,
        ],
        compiler_params=pltpu.CompilerParams(collective_id=0),
    )(x, w_mat)


# baseline (device time: 782348 ns/iter reference)
import jax
import jax.numpy as jnp
from jax import lax
from jax.experimental import pallas as pl
from jax.experimental.pallas import tpu as pltpu

N_DEV = 16


def kernel(x, w_mat):
    m_per, k = x.shape
    _, n_per = w_mat.shape
    m_glob = N_DEV * m_per

    def body(x_ref, w_ref, out_ref, comm_ref, send_sems, recv_sems, credit_sem):
        my = lax.axis_index("i")
        left = (my - 1) % N_DEV
        right = (my + 1) % N_DEV

        comm_ref[0, :, :] = x_ref[:, :]
        out_ref[pl.ds(my * m_per, m_per), :] = jnp.dot(
            x_ref[...], w_ref[...], preferred_element_type=jnp.float32
        )

        for h in range(N_DEV - 1):
            ss = h % 2
            rs = (h + 1) % 2
            if h == 2:
                pl.semaphore_wait(credit_sem, 2)
            elif h > 2:
                pl.semaphore_wait(credit_sem, 1)

            rdma = pltpu.make_async_remote_copy(
                src_ref=comm_ref.at[ss],
                dst_ref=comm_ref.at[rs],
                send_sem=send_sems.at[ss],
                recv_sem=recv_sems.at[rs],
                device_id=(right,),
                device_id_type=pl.DeviceIdType.MESH,
            )
            rdma.start()
            rdma.wait()

            origin = (my - h - 1) % N_DEV
            out_ref[pl.ds(origin * m_per, m_per), :] = jnp.dot(
                comm_ref[rs, :, :], w_ref[...], preferred_element_type=jnp.float32
            )
            if h < N_DEV - 2:
                pl.semaphore_signal(
                    credit_sem,
                    inc=1,
                    device_id=(left,),
                    device_id_type=pl.DeviceIdType.MESH,
                )

    return pl.pallas_call(
        body,
        out_shape=jax.ShapeDtypeStruct((m_glob, n_per), jnp.float32),
        in_specs=[
            pl.BlockSpec(memory_space=pltpu.VMEM),
            pl.BlockSpec(memory_space=pltpu.VMEM),
        ],
        out_specs=pl.BlockSpec(memory_space=pltpu.VMEM),
        scratch_shapes=[
            pltpu.VMEM((2, m_per, k), jnp.float32),
            pltpu.SemaphoreType.DMA((2,)),
            pltpu.SemaphoreType.DMA((2,)),
            pltpu.SemaphoreType.REGULAR,
        ],
    )(x, w_mat)


# device time: 412530 ns/iter; 1.8965x vs baseline; 1.8965x over previous
import jax
import jax.numpy as jnp
from jax import lax
from jax.experimental import pallas as pl
from jax.experimental.pallas import tpu as pltpu

N_DEV = 16
R_HOPS = N_DEV // 2
L_HOPS = N_DEV - 1 - R_HOPS


def kernel(x, w_mat):
    m_per, k = x.shape
    _, n_per = w_mat.shape
    m_glob = N_DEV * m_per

    def body(
        x_ref,
        w_ref,
        out_ref,
        r_buf,
        l_buf,
        r_send_sems,
        r_recv_sems,
        l_send_sems,
        l_recv_sems,
        r_credit,
        l_credit,
    ):
        my = lax.axis_index("i")
        left = (my - 1) % N_DEV
        right = (my + 1) % N_DEV

        def gemm_to(origin, chunk):
            out_ref[pl.ds(origin * m_per, m_per), :] = jnp.dot(
                chunk, w_ref[...], preferred_element_type=jnp.float32
            )

        r_buf[0, :, :] = x_ref[:, :]
        l_buf[0, :, :] = x_ref[:, :]

        for h in range(R_HOPS):
            ss = h % 2
            rs = (h + 1) % 2

            if h == 2:
                pl.semaphore_wait(r_credit, 2)
                pl.semaphore_wait(l_credit, 2)
            elif h > 2:
                pl.semaphore_wait(r_credit, 1)
                if h < L_HOPS:
                    pl.semaphore_wait(l_credit, 1)

            r_rdma = pltpu.make_async_remote_copy(
                src_ref=r_buf.at[ss],
                dst_ref=r_buf.at[rs],
                send_sem=r_send_sems.at[ss],
                recv_sem=r_recv_sems.at[rs],
                device_id=(right,),
                device_id_type=pl.DeviceIdType.MESH,
            )
            r_rdma.start()
            if h < L_HOPS:
                l_rdma = pltpu.make_async_remote_copy(
                    src_ref=l_buf.at[ss],
                    dst_ref=l_buf.at[rs],
                    send_sem=l_send_sems.at[ss],
                    recv_sem=l_recv_sems.at[rs],
                    device_id=(left,),
                    device_id_type=pl.DeviceIdType.MESH,
                )
                l_rdma.start()

            if h == 0:
                gemm_to(my, x_ref[:, :])
            else:
                gemm_to((my - h) % N_DEV, r_buf[ss, :, :])
                gemm_to((my + h) % N_DEV, l_buf[ss, :, :])

            r_rdma.wait()
            if h < L_HOPS:
                l_rdma.wait()

            if h < R_HOPS - 1:
                pl.semaphore_signal(
                    r_credit,
                    inc=1,
                    device_id=(left,),
                    device_id_type=pl.DeviceIdType.MESH,
                )
            if h < L_HOPS - 1:
                pl.semaphore_signal(
                    l_credit,
                    inc=1,
                    device_id=(right,),
                    device_id_type=pl.DeviceIdType.MESH,
                )

        gemm_to((my - R_HOPS) % N_DEV, r_buf[R_HOPS % 2, :, :])

    return pl.pallas_call(
        body,
        out_shape=jax.ShapeDtypeStruct((m_glob, n_per), jnp.float32),
        in_specs=[
            pl.BlockSpec(memory_space=pltpu.VMEM),
            pl.BlockSpec(memory_space=pltpu.VMEM),
        ],
        out_specs=pl.BlockSpec(memory_space=pltpu.VMEM),
        scratch_shapes=[
            pltpu.VMEM((2, m_per, k), jnp.float32),
            pltpu.VMEM((2, m_per, k), jnp.float32),
            pltpu.SemaphoreType.DMA((2,)),
            pltpu.SemaphoreType.DMA((2,)),
            pltpu.SemaphoreType.DMA((2,)),
            pltpu.SemaphoreType.DMA((2,)),
            pltpu.SemaphoreType.REGULAR,
            pltpu.SemaphoreType.REGULAR,
        ],
    )(x, w_mat)


# device time: 394869 ns/iter; 1.9813x vs baseline; 1.0447x over previous
import jax
import jax.numpy as jnp
from jax import lax
from jax.experimental import pallas as pl
from jax.experimental.pallas import tpu as pltpu

N_DEV = 16
R_HOPS = N_DEV // 2
L_HOPS = N_DEV - 1 - R_HOPS
F = 2


def kernel(x, w_mat):
    m_per, k = x.shape
    _, n_per = w_mat.shape
    m_glob = N_DEV * m_per
    m_frag = m_per // F

    def body(
        x_ref,
        w_ref,
        out_ref,
        r_buf,
        l_buf,
        r_send_sems,
        r_recv_sems,
        l_send_sems,
        l_recv_sems,
        r_credit,
        l_credit,
    ):
        my = lax.axis_index("i")
        left = (my - 1) % N_DEV
        right = (my + 1) % N_DEV

        def gemm_frag(origin, f, chunk_frag):
            out_ref[pl.ds(origin * m_per + f * m_frag, m_frag), :] = jnp.dot(
                chunk_frag, w_ref[...], preferred_element_type=jnp.float32
            )

        def desc(buf, sends, recvs, slot_src, slot_dst, f, dev):
            return pltpu.make_async_remote_copy(
                src_ref=buf.at[slot_src, f],
                dst_ref=buf.at[slot_dst, f],
                send_sem=sends.at[slot_src, f],
                recv_sem=recvs.at[slot_dst, f],
                device_id=(dev,),
                device_id_type=pl.DeviceIdType.MESH,
            )

        for h in range(R_HOPS):
            ss = h % 2
            rs = (h + 1) % 2

            if h == 2:
                pl.semaphore_wait(r_credit, 2)
                pl.semaphore_wait(l_credit, 2)
            elif h > 2:
                pl.semaphore_wait(r_credit, 1)
                if h < L_HOPS:
                    pl.semaphore_wait(l_credit, 1)

            r_sends = []
            l_sends = []
            for f in range(F):
                if h >= 1:
                    desc(r_buf, r_send_sems, r_recv_sems, ss, ss, f, left
                         ).wait_recv()
                    desc(l_buf, l_send_sems, l_recv_sems, ss, ss, f, right
                         ).wait_recv()
                if h == 0:
                    rd = pltpu.make_async_remote_copy(
                        src_ref=x_ref.at[pl.ds(f * m_frag, m_frag), :],
                        dst_ref=r_buf.at[rs, f],
                        send_sem=r_send_sems.at[rs, f],
                        recv_sem=r_recv_sems.at[rs, f],
                        device_id=(right,),
                        device_id_type=pl.DeviceIdType.MESH,
                    )
                    ld = pltpu.make_async_remote_copy(
                        src_ref=x_ref.at[pl.ds(f * m_frag, m_frag), :],
                        dst_ref=l_buf.at[rs, f],
                        send_sem=l_send_sems.at[rs, f],
                        recv_sem=l_recv_sems.at[rs, f],
                        device_id=(left,),
                        device_id_type=pl.DeviceIdType.MESH,
                    )
                    rd.start()
                    ld.start()
                    r_sends.append(rd)
                    l_sends.append(ld)
                    gemm_frag(my, f, x_ref[pl.ds(f * m_frag, m_frag), :])
                else:
                    rd = desc(r_buf, r_send_sems, r_recv_sems, ss, rs, f, right)
                    rd.start()
                    r_sends.append(rd)
                    if h < L_HOPS:
                        ld = desc(l_buf, l_send_sems, l_recv_sems, ss, rs, f, left)
                        ld.start()
                        l_sends.append(ld)
                    gemm_frag((my - h) % N_DEV, f, r_buf[ss, f, :, :])
                    gemm_frag((my + h) % N_DEV, f, l_buf[ss, f, :, :])

            for rd in r_sends:
                rd.wait_send()
            for ld in l_sends:
                ld.wait_send()

            if h < R_HOPS - 1:
                pl.semaphore_signal(
                    r_credit,
                    inc=1,
                    device_id=(left,),
                    device_id_type=pl.DeviceIdType.MESH,
                )
            if h < L_HOPS - 1:
                pl.semaphore_signal(
                    l_credit,
                    inc=1,
                    device_id=(right,),
                    device_id_type=pl.DeviceIdType.MESH,
                )

        last = R_HOPS % 2
        for f in range(F):
            desc(r_buf, r_send_sems, r_recv_sems, last, last, f, left
                 ).wait_recv()
            gemm_frag((my - R_HOPS) % N_DEV, f, r_buf[last, f, :, :])

    return pl.pallas_call(
        body,
        out_shape=jax.ShapeDtypeStruct((m_glob, n_per), jnp.float32),
        in_specs=[
            pl.BlockSpec(memory_space=pltpu.VMEM),
            pl.BlockSpec(memory_space=pltpu.VMEM),
        ],
        out_specs=pl.BlockSpec(memory_space=pltpu.VMEM),
        scratch_shapes=[
            pltpu.VMEM((2, F, m_frag, k), jnp.float32),
            pltpu.VMEM((2, F, m_frag, k), jnp.float32),
            pltpu.SemaphoreType.DMA((2, F)),
            pltpu.SemaphoreType.DMA((2, F)),
            pltpu.SemaphoreType.DMA((2, F)),
            pltpu.SemaphoreType.DMA((2, F)),
            pltpu.SemaphoreType.REGULAR,
            pltpu.SemaphoreType.REGULAR,
        ],
    )(x, w_mat)


# device time: 372657 ns/iter; 2.0994x vs baseline; 1.0596x over previous
import jax
import jax.numpy as jnp
from jax import lax
from jax.experimental import pallas as pl
from jax.experimental.pallas import tpu as pltpu

N_DEV = 16
HOPS = N_DEV // 2
F = 2


def kernel(x, w_mat):
    m_per, k = x.shape
    _, n_per = w_mat.shape
    m_glob = N_DEV * m_per
    m_frag = m_per // F

    def body(
        x_ref,
        w_ref,
        out_ref,
        r_buf,
        l_buf,
        r_send_sems,
        r_recv_sems,
        l_send_sems,
        l_recv_sems,
        r_credit,
        l_credit,
    ):
        my = lax.axis_index("i")
        left = (my - 1) % N_DEV
        right = (my + 1) % N_DEV

        def gemm_frag(origin, f, chunk_frag):
            out_ref[pl.ds(origin * m_per + f * m_frag, m_frag), :] = jnp.dot(
                chunk_frag, w_ref[...], preferred_element_type=jnp.float32
            )

        def desc(buf, sends, recvs, slot_src, slot_dst, f, dev):
            return pltpu.make_async_remote_copy(
                src_ref=buf.at[slot_src, f],
                dst_ref=buf.at[slot_dst, f],
                send_sem=sends.at[slot_src, f],
                recv_sem=recvs.at[slot_dst, f],
                device_id=(dev,),
                device_id_type=pl.DeviceIdType.MESH,
            )

        for h in range(HOPS):
            ss = h % 2
            rs = (h + 1) % 2

            if h == 2:
                pl.semaphore_wait(r_credit, 2)
                pl.semaphore_wait(l_credit, 2)
            elif h > 2:
                pl.semaphore_wait(r_credit, 1)
                pl.semaphore_wait(l_credit, 1)

            r_sends = []
            l_sends = []
            for f in range(F):
                if h >= 1:
                    desc(r_buf, r_send_sems, r_recv_sems, ss, ss, f, left
                         ).wait_recv()
                    desc(l_buf, l_send_sems, l_recv_sems, ss, ss, f, right
                         ).wait_recv()
                if h == 0:
                    rd = pltpu.make_async_remote_copy(
                        src_ref=x_ref.at[pl.ds(f * m_frag, m_frag), :],
                        dst_ref=r_buf.at[rs, f],
                        send_sem=r_send_sems.at[rs, f],
                        recv_sem=r_recv_sems.at[rs, f],
                        device_id=(right,),
                        device_id_type=pl.DeviceIdType.MESH,
                    )
                    ld = pltpu.make_async_remote_copy(
                        src_ref=x_ref.at[pl.ds(f * m_frag, m_frag), :],
                        dst_ref=l_buf.at[rs, f],
                        send_sem=l_send_sems.at[rs, f],
                        recv_sem=l_recv_sems.at[rs, f],
                        device_id=(left,),
                        device_id_type=pl.DeviceIdType.MESH,
                    )
                    rd.start()
                    ld.start()
                    r_sends.append(rd)
                    l_sends.append(ld)
                    gemm_frag(my, f, x_ref[pl.ds(f * m_frag, m_frag), :])
                else:
                    if h < HOPS - 1 or f == 0:
                        rd = desc(r_buf, r_send_sems, r_recv_sems, ss, rs, f, right)
                        rd.start()
                        r_sends.append(rd)
                    if h < HOPS - 1 or f == 1:
                        ld = desc(l_buf, l_send_sems, l_recv_sems, ss, rs, f, left)
                        ld.start()
                        l_sends.append(ld)
                    gemm_frag((my - h) % N_DEV, f, r_buf[ss, f, :, :])
                    gemm_frag((my + h) % N_DEV, f, l_buf[ss, f, :, :])

            for rd in r_sends:
                rd.wait_send()
            for ld in l_sends:
                ld.wait_send()

            if h < HOPS - 1:
                pl.semaphore_signal(
                    r_credit,
                    inc=1,
                    device_id=(left,),
                    device_id_type=pl.DeviceIdType.MESH,
                )
                pl.semaphore_signal(
                    l_credit,
                    inc=1,
                    device_id=(right,),
                    device_id_type=pl.DeviceIdType.MESH,
                )

        last = HOPS % 2
        anti = (my + HOPS) % N_DEV
        desc(r_buf, r_send_sems, r_recv_sems, last, last, 0, left).wait_recv()
        gemm_frag(anti, 0, r_buf[last, 0, :, :])
        desc(l_buf, l_send_sems, l_recv_sems, last, last, 1, right).wait_recv()
        gemm_frag(anti, 1, l_buf[last, 1, :, :])

    return pl.pallas_call(
        body,
        out_shape=jax.ShapeDtypeStruct((m_glob, n_per), jnp.float32),
        in_specs=[
            pl.BlockSpec(memory_space=pltpu.VMEM),
            pl.BlockSpec(memory_space=pltpu.VMEM),
        ],
        out_specs=pl.BlockSpec(memory_space=pltpu.VMEM),
        scratch_shapes=[
            pltpu.VMEM((2, F, m_frag, k), jnp.float32),
            pltpu.VMEM((2, F, m_frag, k), jnp.float32),
            pltpu.SemaphoreType.DMA((2, F)),
            pltpu.SemaphoreType.DMA((2, F)),
            pltpu.SemaphoreType.DMA((2, F)),
            pltpu.SemaphoreType.DMA((2, F)),
            pltpu.SemaphoreType.REGULAR,
            pltpu.SemaphoreType.REGULAR,
        ],
    )(x, w_mat)


# device time: 360487 ns/iter; 2.1703x vs baseline; 1.0338x over previous
import jax
import jax.numpy as jnp
from jax import lax
from jax.experimental import pallas as pl
from jax.experimental.pallas import tpu as pltpu

N_DEV = 16
HOPS = N_DEV // 2
F = 2


def kernel(x, w_mat):
    m_per, k = x.shape
    _, n_per = w_mat.shape
    m_glob = N_DEV * m_per
    m_frag = m_per // F

    def body(
        x_ref,
        w_ref,
        out_ref,
        r_buf,
        l_buf,
        r_send_sems,
        r_recv_sems,
        l_send_sems,
        l_recv_sems,
        r_credit,
        l_credit,
    ):
        my = lax.axis_index("i")
        left = (my - 1) % N_DEV
        right = (my + 1) % N_DEV

        def gemm_frag(origin, f, chunk_frag):
            out_ref[pl.ds(origin * m_per + f * m_frag, m_frag), :] = jnp.dot(
                chunk_frag, w_ref[...], preferred_element_type=jnp.float32
            )

        def desc(buf, sends, recvs, slot_src, slot_dst, f, dev):
            return pltpu.make_async_remote_copy(
                src_ref=buf.at[slot_src, f],
                dst_ref=buf.at[slot_dst, f],
                send_sem=sends.at[slot_src, f],
                recv_sem=recvs.at[slot_dst, f],
                device_id=(dev,),
                device_id_type=pl.DeviceIdType.MESH,
            )

        for h in range(HOPS):
            ss = h % 3
            rs = (h + 1) % 3

            if h == 3:
                pl.semaphore_wait(r_credit, 3)
                pl.semaphore_wait(l_credit, 3)
            elif h > 3:
                pl.semaphore_wait(r_credit, 1)
                pl.semaphore_wait(l_credit, 1)

            r_sends = []
            l_sends = []
            for f in range(F):
                if h >= 1:
                    desc(r_buf, r_send_sems, r_recv_sems, ss, ss, f, left
                         ).wait_recv()
                    desc(l_buf, l_send_sems, l_recv_sems, ss, ss, f, right
                         ).wait_recv()
                if h == 0:
                    rd = pltpu.make_async_remote_copy(
                        src_ref=x_ref.at[pl.ds(f * m_frag, m_frag), :],
                        dst_ref=r_buf.at[rs, f],
                        send_sem=r_send_sems.at[rs, f],
                        recv_sem=r_recv_sems.at[rs, f],
                        device_id=(right,),
                        device_id_type=pl.DeviceIdType.MESH,
                    )
                    ld = pltpu.make_async_remote_copy(
                        src_ref=x_ref.at[pl.ds(f * m_frag, m_frag), :],
                        dst_ref=l_buf.at[rs, f],
                        send_sem=l_send_sems.at[rs, f],
                        recv_sem=l_recv_sems.at[rs, f],
                        device_id=(left,),
                        device_id_type=pl.DeviceIdType.MESH,
                    )
                    rd.start()
                    ld.start()
                    r_sends.append(rd)
                    l_sends.append(ld)
                    gemm_frag(my, f, x_ref[pl.ds(f * m_frag, m_frag), :])
                else:
                    if h < HOPS - 1 or f == 0:
                        rd = desc(r_buf, r_send_sems, r_recv_sems, ss, rs, f, right)
                        rd.start()
                        r_sends.append(rd)
                    if h < HOPS - 1 or f == 1:
                        ld = desc(l_buf, l_send_sems, l_recv_sems, ss, rs, f, left)
                        ld.start()
                        l_sends.append(ld)
                    gemm_frag((my - h) % N_DEV, f, r_buf[ss, f, :, :])
                    gemm_frag((my + h) % N_DEV, f, l_buf[ss, f, :, :])

            if h < HOPS - 1:
                pl.semaphore_signal(
                    r_credit,
                    inc=1,
                    device_id=(left,),
                    device_id_type=pl.DeviceIdType.MESH,
                )
                pl.semaphore_signal(
                    l_credit,
                    inc=1,
                    device_id=(right,),
                    device_id_type=pl.DeviceIdType.MESH,
                )

            for rd in r_sends:
                rd.wait_send()
            for ld in l_sends:
                ld.wait_send()

        last = HOPS % 3
        anti = (my + HOPS) % N_DEV
        desc(r_buf, r_send_sems, r_recv_sems, last, last, 0, left).wait_recv()
        gemm_frag(anti, 0, r_buf[last, 0, :, :])
        desc(l_buf, l_send_sems, l_recv_sems, last, last, 1, right).wait_recv()
        gemm_frag(anti, 1, l_buf[last, 1, :, :])

    return pl.pallas_call(
        body,
        out_shape=jax.ShapeDtypeStruct((m_glob, n_per), jnp.float32),
        in_specs=[
            pl.BlockSpec(memory_space=pltpu.VMEM),
            pl.BlockSpec(memory_space=pltpu.VMEM),
        ],
        out_specs=pl.BlockSpec(memory_space=pltpu.VMEM),
        scratch_shapes=[
            pltpu.VMEM((3, F, m_frag, k), jnp.float32),
            pltpu.VMEM((3, F, m_frag, k), jnp.float32),
            pltpu.SemaphoreType.DMA((3, F)),
            pltpu.SemaphoreType.DMA((3, F)),
            pltpu.SemaphoreType.DMA((3, F)),
            pltpu.SemaphoreType.DMA((3, F)),
            pltpu.SemaphoreType.REGULAR,
            pltpu.SemaphoreType.REGULAR,
        ],
    )(x, w_mat)
